# Initial kernel scaffold; baseline (speedup 1.0000x reference)
import jax
import jax.numpy as jnp
from jax.experimental import pallas as pl


def kernel(x, edge_index, W1, b1, W_mu, b_mu, W_ls, b_ls):
    raise NotImplementedError("write your pallas kernel here")


# speedup vs baseline: 9.6368x; 9.6368x over previous
import functools

import jax
import jax.numpy as jnp
from jax import lax
from jax.experimental import pallas as pl
from jax.experimental.pallas import tpu as pltpu
from jax.experimental.pallas import tpu_sc as plsc

N = 10000
D = 128
OUT = 64
NC = 2
NS = 16
NW = NC * NS
CHUNK = 128
CPW = 80
PH = 2
SLAB = CPW // PH
E_PAD = NW * CPW * CHUNK
RPT = 632
ACC_ROWS = NS * RPT

_MESH = plsc.VectorSubcoreMesh(
    core_axis_name="c", subcore_axis_name="s", num_cores=NC, num_subcores=NS)


@functools.partial(
    pl.kernel,
    out_type=jax.ShapeDtypeStruct((NC, ACC_ROWS, D), jnp.float32),
    mesh=_MESH,
    scratch_types=[
        pltpu.VMEM((SLAB, CHUNK), jnp.int32),
        pltpu.VMEM((CHUNK, D), jnp.float32),
        pltpu.VMEM_SHARED((ACC_ROWS, D), jnp.float32),
    ],
)
def _sc_degree(dst_hbm, ones_hbm, zeros_hbm, out_hbm, dstv, onesv, degsp):
    c = lax.axis_index("c")
    s = lax.axis_index("s")
    wid = s * NC + c
    pltpu.sync_copy(ones_hbm, onesv)
    pltpu.sync_copy(zeros_hbm, degsp.at[pl.ds(s * RPT, RPT)])
    plsc.subcore_barrier()

    for p in range(PH):
        pltpu.sync_copy(dst_hbm.at[pl.ds(wid * CPW + p * SLAB, SLAB)], dstv)

        def body(j, _):
            pltpu.sync_copy(onesv, degsp.at[dstv.at[j]], add=True)
            return 0

        lax.fori_loop(0, SLAB, body, 0)
    plsc.subcore_barrier()
    pltpu.sync_copy(degsp.at[pl.ds(s * RPT, RPT)],
                    out_hbm.at[c].at[pl.ds(s * RPT, RPT)])


@functools.partial(
    pl.kernel,
    out_type=jax.ShapeDtypeStruct((NC, ACC_ROWS, D), jnp.float32),
    mesh=_MESH,
    scratch_types=[
        pltpu.VMEM((SLAB, CHUNK), jnp.int32),
        pltpu.VMEM((SLAB, CHUNK), jnp.int32),
        pltpu.VMEM((CHUNK, D), jnp.float32),
        pltpu.VMEM((CHUNK, D), jnp.float32),
        pltpu.VMEM_SHARED((ACC_ROWS, D), jnp.float32),
        pltpu.SemaphoreType.DMA,
        pltpu.SemaphoreType.DMA,
    ],
)
def _sc_propagate(g_hbm, src_hbm, dst_hbm, zeros_hbm, out_hbm,
                  srcv, dstv, bufa, bufb, acc, sema, semb):
    c = lax.axis_index("c")
    s = lax.axis_index("s")
    wid = s * NC + c
    pltpu.sync_copy(zeros_hbm, acc.at[pl.ds(s * RPT, RPT)])
    plsc.subcore_barrier()

    for p in range(PH):
        base = wid * CPW + p * SLAB
        pltpu.sync_copy(src_hbm.at[pl.ds(base, SLAB)], srcv)
        pltpu.sync_copy(dst_hbm.at[pl.ds(base, SLAB)], dstv)

        pltpu.async_copy(g_hbm.at[srcv.at[0]], bufa, sema)

        def body(j, _):
            pltpu.make_async_copy(g_hbm.at[srcv.at[j]], bufa, sema).wait()
            pltpu.async_copy(g_hbm.at[srcv.at[j + 1]], bufb, semb)
            pltpu.sync_copy(bufa, acc.at[dstv.at[j]], add=True)
            pltpu.make_async_copy(g_hbm.at[srcv.at[j + 1]], bufb, semb).wait()

            @pl.when(j + 2 < SLAB)
            def _():
                pltpu.async_copy(g_hbm.at[srcv.at[j + 2]], bufa, sema)

            pltpu.sync_copy(bufb, acc.at[dstv.at[j + 1]], add=True)
            return 0

        lax.fori_loop(0, SLAB // 2, lambda i, carry: body(i * 2, carry), 0,
                      unroll=False)
    plsc.subcore_barrier()
    pltpu.sync_copy(acc.at[pl.ds(s * RPT, RPT)],
                    out_hbm.at[c].at[pl.ds(s * RPT, RPT)])


_RB = 1000


def _tc1_body(deg_ref, x_ref, w1_ref, g1_ref, dinv_ref):
    deg = deg_ref[0, :, 0:1] + deg_ref[1, :, 0:1] + 1.0
    dinv = lax.rsqrt(deg)
    dinv_ref[...] = dinv
    g1_ref[...] = jnp.dot(x_ref[...], w1_ref[...],
                          preferred_element_type=jnp.float32) * dinv


def _tc1(degp, x, w1):
    return pl.pallas_call(
        _tc1_body,
        grid=(N // _RB,),
        in_specs=[
            pl.BlockSpec((NC, _RB, D), lambda i: (0, i, 0)),
            pl.BlockSpec((_RB, D), lambda i: (i, 0)),
            pl.BlockSpec((D, D), lambda i: (0, 0)),
        ],
        out_specs=[
            pl.BlockSpec((_RB, D), lambda i: (i, 0)),
            pl.BlockSpec((_RB, 1), lambda i: (i, 0)),
        ],
        out_shape=[
            jax.ShapeDtypeStruct((N, D), jnp.float32),
            jax.ShapeDtypeStruct((N, 1), jnp.float32),
        ],
    )(degp, x, w1)


def _tc2_body(p1a_ref, p1b_ref, g1_ref, dinv_ref, b1_ref, wcat_ref, g2_ref):
    dinv = dinv_ref[...]
    z = jnp.maximum(
        dinv * (p1a_ref[...] + p1b_ref[...] + g1_ref[...]) + b1_ref[...], 0.0)
    g2_ref[...] = jnp.dot(z, wcat_ref[...],
                          preferred_element_type=jnp.float32) * dinv


def _tc2(p1a, p1b, g1, dinv, b1, wcat):
    return pl.pallas_call(
        _tc2_body,
        grid=(N // _RB,),
        in_specs=[
            pl.BlockSpec((_RB, D), lambda i: (i, 0)),
            pl.BlockSpec((_RB, D), lambda i: (i, 0)),
            pl.BlockSpec((_RB, D), lambda i: (i, 0)),
            pl.BlockSpec((_RB, 1), lambda i: (i, 0)),
            pl.BlockSpec((1, D), lambda i: (0, 0)),
            pl.BlockSpec((D, D), lambda i: (0, 0)),
        ],
        out_specs=pl.BlockSpec((_RB, D), lambda i: (i, 0)),
        out_shape=jax.ShapeDtypeStruct((N, D), jnp.float32),
    )(p1a, p1b, g1, dinv, b1, wcat)


def _tc3_body(p2a_ref, p2b_ref, g2_ref, dinv_ref, bcat_ref, out_ref):
    out_ref[...] = (dinv_ref[...] *
                    (p2a_ref[...] + p2b_ref[...] + g2_ref[...]) +
                    bcat_ref[...])


def _tc3(p2a, p2b, g2, dinv, bcat):
    return pl.pallas_call(
        _tc3_body,
        grid=(N // _RB,),
        in_specs=[
            pl.BlockSpec((_RB, D), lambda i: (i, 0)),
            pl.BlockSpec((_RB, D), lambda i: (i, 0)),
            pl.BlockSpec((_RB, D), lambda i: (i, 0)),
            pl.BlockSpec((_RB, 1), lambda i: (i, 0)),
            pl.BlockSpec((1, D), lambda i: (0, 0)),
        ],
        out_specs=pl.BlockSpec((_RB, D), lambda i: (i, 0)),
        out_shape=jax.ShapeDtypeStruct((N, D), jnp.float32),
    )(p2a, p2b, g2, dinv, bcat)


def kernel(x, edge_index, W1, b1, W_mu, b_mu, W_ls, b_ls):
    E = edge_index.shape[1]
    pad = E_PAD - E
    src2d = jnp.concatenate(
        [edge_index[0], jnp.zeros((pad,), jnp.int32)]).reshape(NW * CPW, CHUNK)
    dst2d = jnp.concatenate(
        [edge_index[1], jnp.full((pad,), N, jnp.int32)]).reshape(NW * CPW, CHUNK)

    onesD = jnp.ones((CHUNK, D), jnp.float32)
    zerosD = jnp.zeros((RPT, D), jnp.float32)
    wcat = jnp.concatenate([W_mu, W_ls], axis=1)
    bcat = jnp.concatenate([b_mu, b_ls]).reshape(1, D)

    degp = _sc_degree(dst2d, onesD, zerosD)
    g1, dinv = _tc1(degp, x, W1)
    p1 = _sc_propagate(g1, src2d, dst2d, zerosD)
    g2 = _tc2(p1[0, :N], p1[1, :N], g1, dinv, b1.reshape(1, D), wcat)
    p2 = _sc_propagate(g2, src2d, dst2d, zerosD)
    out2 = _tc3(p2[0, :N], p2[1, :N], g2, dinv, bcat)
    return (out2[:, :OUT], out2[:, OUT:])


# speedup vs baseline: 9.7803x; 1.0149x over previous
import functools

import jax
import jax.numpy as jnp
from jax import lax
from jax.experimental import pallas as pl
from jax.experimental.pallas import tpu as pltpu
from jax.experimental.pallas import tpu_sc as plsc

N = 10000
D = 128
OUT = 64
NC = 2
NS = 16
NW = NC * NS
CHUNK = 64
CPW = 160
PH = 4
SLAB = CPW // PH
NBUF = 4
E_PAD = NW * CPW * CHUNK
RPT = 632
ACC_ROWS = NS * RPT

_MESH = plsc.VectorSubcoreMesh(
    core_axis_name="c", subcore_axis_name="s", num_cores=NC, num_subcores=NS)


@functools.partial(
    pl.kernel,
    out_type=jax.ShapeDtypeStruct((NC, ACC_ROWS, D), jnp.float32),
    mesh=_MESH,
    scratch_types=[
        pltpu.VMEM((SLAB, CHUNK), jnp.int32),
        pltpu.VMEM((CHUNK, D), jnp.float32),
        pltpu.VMEM_SHARED((ACC_ROWS, D), jnp.float32),
    ] + [pltpu.SemaphoreType.DMA] * NBUF,
)
def _sc_degree(dst_hbm, ones_hbm, zeros_hbm, out_hbm, dstv, onesv, degsp,
               *sems):
    c = lax.axis_index("c")
    s = lax.axis_index("s")
    wid = s * NC + c
    pltpu.sync_copy(ones_hbm, onesv)
    pltpu.sync_copy(zeros_hbm, degsp.at[pl.ds(s * RPT, RPT)])
    plsc.subcore_barrier()

    for p in range(PH):
        pltpu.sync_copy(dst_hbm.at[pl.ds(wid * CPW + p * SLAB, SLAB)], dstv)

        def body(g, _):
            for b in range(NBUF):
                pltpu.async_copy(onesv, degsp.at[dstv.at[g * NBUF + b]],
                                 sems[b], add=True)
            for b in range(NBUF):
                pltpu.make_async_copy(
                    onesv, degsp.at[dstv.at[g * NBUF + b]], sems[b]).wait()
            return 0

        lax.fori_loop(0, SLAB // NBUF, body, 0)
    plsc.subcore_barrier()
    pltpu.sync_copy(degsp.at[pl.ds(s * RPT, RPT)],
                    out_hbm.at[c].at[pl.ds(s * RPT, RPT)])


@functools.partial(
    pl.kernel,
    out_type=jax.ShapeDtypeStruct((NC, ACC_ROWS, D), jnp.float32),
    mesh=_MESH,
    scratch_types=[
        pltpu.VMEM((SLAB, CHUNK), jnp.int32),
        pltpu.VMEM((SLAB, CHUNK), jnp.int32),
        [pltpu.VMEM((CHUNK, D), jnp.float32)] * NBUF,
        pltpu.VMEM_SHARED((ACC_ROWS, D), jnp.float32),
        [pltpu.SemaphoreType.DMA] * NBUF,
        [pltpu.SemaphoreType.DMA] * NBUF,
    ],
)
def _sc_propagate(g_hbm, src_hbm, dst_hbm, zeros_hbm, out_hbm,
                  srcv, dstv, bufs, acc, gsems, ssems):
    c = lax.axis_index("c")
    s = lax.axis_index("s")
    wid = s * NC + c
    pltpu.sync_copy(zeros_hbm, acc.at[pl.ds(s * RPT, RPT)])
    plsc.subcore_barrier()

    for p in range(PH):
        base = wid * CPW + p * SLAB
        pltpu.sync_copy(src_hbm.at[pl.ds(base, SLAB)], srcv)
        pltpu.sync_copy(dst_hbm.at[pl.ds(base, SLAB)], dstv)

        for b in range(NBUF):
            pltpu.async_copy(g_hbm.at[srcv.at[b]], bufs[b], gsems[b])

        def group(g, _):
            for b in range(NBUF):
                j = g * NBUF + b
                pltpu.make_async_copy(g_hbm.at[srcv.at[j]], bufs[b],
                                      gsems[b]).wait()
                pltpu.async_copy(bufs[b], acc.at[dstv.at[j]], ssems[b],
                                 add=True)

                @pl.when(j + NBUF < SLAB)
                def _():
                    pltpu.make_async_copy(bufs[b], acc.at[dstv.at[j]],
                                          ssems[b]).wait()
                    pltpu.async_copy(g_hbm.at[srcv.at[j + NBUF]], bufs[b],
                                     gsems[b])
            return 0

        lax.fori_loop(0, SLAB // NBUF, group, 0)
        for b in range(NBUF):
            j = SLAB - NBUF + b
            pltpu.make_async_copy(bufs[b], acc.at[dstv.at[j]],
                                  ssems[b]).wait()
    plsc.subcore_barrier()
    pltpu.sync_copy(acc.at[pl.ds(s * RPT, RPT)],
                    out_hbm.at[c].at[pl.ds(s * RPT, RPT)])


_RB = 1000


def _tc1_body(deg_ref, x_ref, w1_ref, g1_ref, dinv_ref):
    deg = deg_ref[0, :, 0:1] + deg_ref[1, :, 0:1] + 1.0
    dinv = lax.rsqrt(deg)
    dinv_ref[...] = dinv
    g1_ref[...] = jnp.dot(x_ref[...], w1_ref[...],
                          preferred_element_type=jnp.float32) * dinv


def _tc1(degp, x, w1):
    return pl.pallas_call(
        _tc1_body,
        grid=(N // _RB,),
        in_specs=[
            pl.BlockSpec((NC, _RB, D), lambda i: (0, i, 0)),
            pl.BlockSpec((_RB, D), lambda i: (i, 0)),
            pl.BlockSpec((D, D), lambda i: (0, 0)),
        ],
        out_specs=[
            pl.BlockSpec((_RB, D), lambda i: (i, 0)),
            pl.BlockSpec((_RB, 1), lambda i: (i, 0)),
        ],
        out_shape=[
            jax.ShapeDtypeStruct((N, D), jnp.float32),
            jax.ShapeDtypeStruct((N, 1), jnp.float32),
        ],
    )(degp, x, w1)


def _tc2_body(p1a_ref, p1b_ref, g1_ref, dinv_ref, b1_ref, wcat_ref, g2_ref):
    dinv = dinv_ref[...]
    z = jnp.maximum(
        dinv * (p1a_ref[...] + p1b_ref[...] + g1_ref[...]) + b1_ref[...], 0.0)
    g2_ref[...] = jnp.dot(z, wcat_ref[...],
                          preferred_element_type=jnp.float32) * dinv


def _tc2(p1a, p1b, g1, dinv, b1, wcat):
    return pl.pallas_call(
        _tc2_body,
        grid=(N // _RB,),
        in_specs=[
            pl.BlockSpec((_RB, D), lambda i: (i, 0)),
            pl.BlockSpec((_RB, D), lambda i: (i, 0)),
            pl.BlockSpec((_RB, D), lambda i: (i, 0)),
            pl.BlockSpec((_RB, 1), lambda i: (i, 0)),
            pl.BlockSpec((1, D), lambda i: (0, 0)),
            pl.BlockSpec((D, D), lambda i: (0, 0)),
        ],
        out_specs=pl.BlockSpec((_RB, D), lambda i: (i, 0)),
        out_shape=jax.ShapeDtypeStruct((N, D), jnp.float32),
    )(p1a, p1b, g1, dinv, b1, wcat)


def _tc3_body(p2a_ref, p2b_ref, g2_ref, dinv_ref, bcat_ref, out_ref):
    out_ref[...] = (dinv_ref[...] *
                    (p2a_ref[...] + p2b_ref[...] + g2_ref[...]) +
                    bcat_ref[...])


def _tc3(p2a, p2b, g2, dinv, bcat):
    return pl.pallas_call(
        _tc3_body,
        grid=(N // _RB,),
        in_specs=[
            pl.BlockSpec((_RB, D), lambda i: (i, 0)),
            pl.BlockSpec((_RB, D), lambda i: (i, 0)),
            pl.BlockSpec((_RB, D), lambda i: (i, 0)),
            pl.BlockSpec((_RB, 1), lambda i: (i, 0)),
            pl.BlockSpec((1, D), lambda i: (0, 0)),
        ],
        out_specs=pl.BlockSpec((_RB, D), lambda i: (i, 0)),
        out_shape=jax.ShapeDtypeStruct((N, D), jnp.float32),
    )(p2a, p2b, g2, dinv, bcat)


def kernel(x, edge_index, W1, b1, W_mu, b_mu, W_ls, b_ls):
    E = edge_index.shape[1]
    pad = E_PAD - E
    src2d = jnp.concatenate(
        [edge_index[0], jnp.zeros((pad,), jnp.int32)]).reshape(NW * CPW, CHUNK)
    dst2d = jnp.concatenate(
        [edge_index[1], jnp.full((pad,), N, jnp.int32)]).reshape(NW * CPW, CHUNK)

    onesD = jnp.ones((CHUNK, D), jnp.float32)
    zerosD = jnp.zeros((RPT, D), jnp.float32)
    wcat = jnp.concatenate([W_mu, W_ls], axis=1)
    bcat = jnp.concatenate([b_mu, b_ls]).reshape(1, D)

    degp = _sc_degree(dst2d, onesD, zerosD)
    g1, dinv = _tc1(degp, x, W1)
    p1 = _sc_propagate(g1, src2d, dst2d, zerosD)
    g2 = _tc2(p1[0, :N], p1[1, :N], g1, dinv, b1.reshape(1, D), wcat)
    p2 = _sc_propagate(g2, src2d, dst2d, zerosD)
    out2 = _tc3(p2[0, :N], p2[1, :N], g2, dinv, bcat)
    return (out2[:, :OUT], out2[:, OUT:])


# speedup vs baseline: 10.2446x; 1.0475x over previous
import functools

import jax
import jax.numpy as jnp
from jax import lax
from jax.experimental import pallas as pl
from jax.experimental.pallas import tpu as pltpu
from jax.experimental.pallas import tpu_sc as plsc

N = 10000
D = 128
OUT = 64
NC = 2
NS = 16
NW = NC * NS
CHUNK = 64
CPW = 160
PH = 4
SLAB = CPW // PH
NBUF = 4
RPT = 632
ACC_ROWS = NS * RPT
E_PAD = NW * CPW * CHUNK

_MESH = plsc.VectorSubcoreMesh(
    core_axis_name="c", subcore_axis_name="s", num_cores=NC, num_subcores=NS)


@functools.partial(
    pl.kernel,
    out_type=jax.ShapeDtypeStruct((NC, ACC_ROWS, D), jnp.float32),
    mesh=_MESH,
    scratch_types=[
        pltpu.VMEM((CPW, CHUNK), jnp.int32),
        pltpu.VMEM((CHUNK, D), jnp.float32),
        pltpu.VMEM_SHARED((ACC_ROWS, D), jnp.float32),
        [pltpu.SemaphoreType.DMA] * NBUF,
    ],
)
def _sc_degree(dst_hbm, ones_hbm, zeros_hbm, out_hbm, dstv, onesv, degsp,
               sems):
    c = lax.axis_index("c")
    s = lax.axis_index("s")
    wid = s * NC + c
    pltpu.sync_copy(dst_hbm.at[pl.ds(wid * CPW, CPW)], dstv)
    pltpu.sync_copy(ones_hbm, onesv)
    pltpu.sync_copy(zeros_hbm, degsp.at[pl.ds(s * RPT, RPT)])
    plsc.subcore_barrier()

    def body(g, _):
        for b in range(NBUF):
            pltpu.async_copy(onesv, degsp.at[dstv.at[g * NBUF + b]],
                             sems[b], add=True)
        for b in range(NBUF):
            pltpu.make_async_copy(
                onesv, degsp.at[dstv.at[g * NBUF + b]], sems[b]).wait()
        return 0

    lax.fori_loop(0, CPW // NBUF, body, 0)
    plsc.subcore_barrier()
    pltpu.sync_copy(degsp.at[pl.ds(s * RPT, RPT)],
                    out_hbm.at[c].at[pl.ds(s * RPT, RPT)])


@functools.partial(
    pl.kernel,
    out_type=jax.ShapeDtypeStruct((NC, ACC_ROWS, D), jnp.float32),
    mesh=_MESH,
    scratch_types=[
        pltpu.VMEM((SLAB, CHUNK), jnp.int32),
        pltpu.VMEM((SLAB, CHUNK), jnp.int32),
        [pltpu.VMEM((CHUNK, D), jnp.float32)] * NBUF,
        pltpu.VMEM_SHARED((ACC_ROWS, D), jnp.float32),
        [pltpu.SemaphoreType.DMA] * NBUF,
        [pltpu.SemaphoreType.DMA] * NBUF,
    ],
)
def _sc_propagate(g_hbm, src_hbm, dst_hbm, zeros_hbm, out_hbm,
                  srcv, dstv, bufs, acc, gsems, ssems):
    c = lax.axis_index("c")
    s = lax.axis_index("s")
    wid = s * NC + c
    pltpu.sync_copy(zeros_hbm, acc.at[pl.ds(s * RPT, RPT)])
    plsc.subcore_barrier()

    for p in range(PH):
        base = wid * CPW + p * SLAB
        pltpu.sync_copy(src_hbm.at[pl.ds(base, SLAB)], srcv)
        pltpu.sync_copy(dst_hbm.at[pl.ds(base, SLAB)], dstv)

        for b in range(NBUF):
            pltpu.async_copy(g_hbm.at[srcv.at[b]], bufs[b], gsems[b])

        def group(g, _):
            for b in range(NBUF):
                j = g * NBUF + b
                pltpu.make_async_copy(g_hbm.at[srcv.at[j]], bufs[b],
                                      gsems[b]).wait()
                pltpu.async_copy(bufs[b], acc.at[dstv.at[j]], ssems[b],
                                 add=True)

                @pl.when(j + NBUF < SLAB)
                def _():
                    pltpu.make_async_copy(bufs[b], acc.at[dstv.at[j]],
                                          ssems[b]).wait()
                    pltpu.async_copy(g_hbm.at[srcv.at[j + NBUF]], bufs[b],
                                     gsems[b])
            return 0

        lax.fori_loop(0, SLAB // NBUF, group, 0)
        for b in range(NBUF):
            j = SLAB - NBUF + b
            pltpu.make_async_copy(bufs[b], acc.at[dstv.at[j]],
                                  ssems[b]).wait()
    plsc.subcore_barrier()
    pltpu.sync_copy(acc.at[pl.ds(s * RPT, RPT)],
                    out_hbm.at[c].at[pl.ds(s * RPT, RPT)])


_RB = 1000


def _tc1_body(deg_ref, x_ref, w1_ref, g1_ref, dinv_ref):
    deg = deg_ref[0, :, 0:1] + deg_ref[1, :, 0:1] + 1.0
    dinv = lax.rsqrt(deg)
    dinv_ref[...] = dinv
    g1_ref[...] = jnp.dot(x_ref[...], w1_ref[...],
                          preferred_element_type=jnp.float32) * dinv


def _tc1(degp, x, w1):
    return pl.pallas_call(
        _tc1_body,
        grid=(N // _RB,),
        in_specs=[
            pl.BlockSpec((NC, _RB, D), lambda i: (0, i, 0)),
            pl.BlockSpec((_RB, D), lambda i: (i, 0)),
            pl.BlockSpec((D, D), lambda i: (0, 0)),
        ],
        out_specs=[
            pl.BlockSpec((_RB, D), lambda i: (i, 0)),
            pl.BlockSpec((_RB, 1), lambda i: (i, 0)),
        ],
        out_shape=[
            jax.ShapeDtypeStruct((N, D), jnp.float32),
            jax.ShapeDtypeStruct((N, 1), jnp.float32),
        ],
    )(degp, x, w1)


def _tc2_body(p1_ref, g1_ref, dinv_ref, b1_ref, wcat_ref, g2_ref):
    dinv = dinv_ref[...]
    z = jnp.maximum(
        dinv * (p1_ref[0] + p1_ref[1] + g1_ref[...]) + b1_ref[...], 0.0)
    g2_ref[...] = jnp.dot(z, wcat_ref[...],
                          preferred_element_type=jnp.float32) * dinv


def _tc2(p1, g1, dinv, b1, wcat):
    return pl.pallas_call(
        _tc2_body,
        grid=(N // _RB,),
        in_specs=[
            pl.BlockSpec((NC, _RB, D), lambda i: (0, i, 0)),
            pl.BlockSpec((_RB, D), lambda i: (i, 0)),
            pl.BlockSpec((_RB, 1), lambda i: (i, 0)),
            pl.BlockSpec((1, D), lambda i: (0, 0)),
            pl.BlockSpec((D, D), lambda i: (0, 0)),
        ],
        out_specs=pl.BlockSpec((_RB, D), lambda i: (i, 0)),
        out_shape=jax.ShapeDtypeStruct((N, D), jnp.float32),
    )(p1, g1, dinv, b1, wcat)


def _tc3_body(p2_ref, g2_ref, dinv_ref, bcat_ref, out_ref):
    out_ref[...] = (dinv_ref[...] *
                    (p2_ref[0] + p2_ref[1] + g2_ref[...]) +
                    bcat_ref[...])


def _tc3(p2, g2, dinv, bcat):
    return pl.pallas_call(
        _tc3_body,
        grid=(N // _RB,),
        in_specs=[
            pl.BlockSpec((NC, _RB, D), lambda i: (0, i, 0)),
            pl.BlockSpec((_RB, D), lambda i: (i, 0)),
            pl.BlockSpec((_RB, 1), lambda i: (i, 0)),
            pl.BlockSpec((1, D), lambda i: (0, 0)),
        ],
        out_specs=pl.BlockSpec((_RB, D), lambda i: (i, 0)),
        out_shape=jax.ShapeDtypeStruct((N, D), jnp.float32),
    )(p2, g2, dinv, bcat)


def kernel(x, edge_index, W1, b1, W_mu, b_mu, W_ls, b_ls):
    E = edge_index.shape[1]
    pad = E_PAD - E
    src2d = jnp.concatenate(
        [edge_index[0], jnp.zeros((pad,), jnp.int32)]).reshape(NW * CPW, CHUNK)
    dst2d = jnp.concatenate(
        [edge_index[1], jnp.full((pad,), N, jnp.int32)]).reshape(NW * CPW, CHUNK)

    onesD = jnp.ones((CHUNK, D), jnp.float32)
    zerosD = jnp.zeros((RPT, D), jnp.float32)
    wcat = jnp.concatenate([W_mu, W_ls], axis=1)
    bcat = jnp.concatenate([b_mu, b_ls]).reshape(1, D)

    degp = _sc_degree(dst2d, onesD, zerosD)
    g1, dinv = _tc1(degp, x, W1)
    p1 = _sc_propagate(g1, src2d, dst2d, zerosD)
    g2 = _tc2(p1[:, :N], g1, dinv, b1.reshape(1, D), wcat)
    p2 = _sc_propagate(g2, src2d, dst2d, zerosD)
    out2 = _tc3(p2[:, :N], g2, dinv, bcat)
    return (out2[:, :OUT], out2[:, OUT:])


# speedup vs baseline: 14.1228x; 1.3786x over previous
import functools

import jax
import jax.numpy as jnp
from jax import lax
from jax.experimental import pallas as pl
from jax.experimental.pallas import tpu as pltpu
from jax.experimental.pallas import tpu_sc as plsc

N = 10000
D = 128
OUT = 64
NC = 2
NS = 16
NW = NC * NS
CHUNK = 64
CPW = 160
NBUF = 4
RPT = 632
ACC_ROWS = NS * RPT
E_PAD = NW * CPW * CHUNK

PCHUNK = 16
PCPW = E_PAD // (NW * PCHUNK)
PSLAB = 8
PNPH = PCPW // PSLAB
PNBUF = 2
HN = 5000
PAR = 5120
PAPT = PAR // NS
TRASH = 5100
GT = 10

_MESH = plsc.VectorSubcoreMesh(
    core_axis_name="c", subcore_axis_name="s", num_cores=NC, num_subcores=NS)


@functools.partial(
    pl.kernel,
    out_type=jax.ShapeDtypeStruct((NC, ACC_ROWS, D), jnp.float32),
    mesh=_MESH,
    scratch_types=[
        pltpu.VMEM((CPW, CHUNK), jnp.int32),
        pltpu.VMEM((CHUNK, D), jnp.float32),
        pltpu.VMEM_SHARED((ACC_ROWS, D), jnp.float32),
        [pltpu.SemaphoreType.DMA] * NBUF,
    ],
)
def _sc_degree(dst_hbm, ones_hbm, zeros_hbm, out_hbm, dstv, onesv, degsp,
               sems):
    c = lax.axis_index("c")
    s = lax.axis_index("s")
    wid = s * NC + c
    pltpu.sync_copy(dst_hbm.at[pl.ds(wid * CPW, CPW)], dstv)
    pltpu.sync_copy(ones_hbm, onesv)
    pltpu.sync_copy(zeros_hbm, degsp.at[pl.ds(s * RPT, RPT)])
    plsc.subcore_barrier()

    def body(g, _):
        for b in range(NBUF):
            pltpu.async_copy(onesv, degsp.at[dstv.at[g * NBUF + b]],
                             sems[b], add=True)
        for b in range(NBUF):
            pltpu.make_async_copy(
                onesv, degsp.at[dstv.at[g * NBUF + b]], sems[b]).wait()
        return 0

    lax.fori_loop(0, CPW // NBUF, body, 0)
    plsc.subcore_barrier()
    pltpu.sync_copy(degsp.at[pl.ds(s * RPT, RPT)],
                    out_hbm.at[c].at[pl.ds(s * RPT, RPT)])


@functools.partial(
    pl.kernel,
    out_type=jax.ShapeDtypeStruct((2, NC, PAR, D), jnp.float32),
    mesh=_MESH,
    scratch_types=[
        [pltpu.VMEM((PSLAB, PCHUNK), jnp.int32)] * 2,
        [pltpu.VMEM((PSLAB, PCHUNK), jnp.int32)] * 2,
        [pltpu.VMEM((PCHUNK, D), jnp.float32)] * PNBUF,
        pltpu.VMEM_SHARED((N, D), jnp.float32),
        pltpu.VMEM_SHARED((PAR, D), jnp.float32),
        [pltpu.SemaphoreType.DMA] * PNBUF,
        [pltpu.SemaphoreType.DMA] * PNBUF,
        [pltpu.SemaphoreType.DMA] * 2,
    ],
)
def _sc_propagate(g_hbm, src_hbm, dst0_hbm, dst1_hbm, zeros_hbm, out_hbm,
                  srcv, dstv, bufs, gsp, acc, gsems, ssems, slsems):
    c = lax.axis_index("c")
    s = lax.axis_index("s")
    wid = s * NC + c
    base = wid * PCPW

    @pl.when(s < GT)
    def _():
        pltpu.sync_copy(g_hbm.at[pl.ds(s * (N // GT), N // GT)],
                        gsp.at[pl.ds(s * (N // GT), N // GT)])

    for h in range(2):
        dst_hbm = dst0_hbm if h == 0 else dst1_hbm
        pltpu.sync_copy(zeros_hbm.at[pl.ds(0, PAPT)],
                        acc.at[pl.ds(s * PAPT, PAPT)])
        plsc.subcore_barrier()

        pltpu.async_copy(src_hbm.at[pl.ds(base, PSLAB)], srcv[0], slsems[0])
        pltpu.async_copy(dst_hbm.at[pl.ds(base, PSLAB)], dstv[0], slsems[0])

        def phase_pair(pp, _):
            for q in range(2):
                ph = pp * 2 + q
                off = base + ph * PSLAB
                pltpu.make_async_copy(src_hbm.at[pl.ds(off, PSLAB)],
                                      srcv[q], slsems[q]).wait()
                pltpu.make_async_copy(dst_hbm.at[pl.ds(off, PSLAB)],
                                      dstv[q], slsems[q]).wait()

                @pl.when(ph + 1 < PNPH)
                def _():
                    noff = base + (ph + 1) * PSLAB
                    pltpu.async_copy(src_hbm.at[pl.ds(noff, PSLAB)],
                                     srcv[1 - q], slsems[1 - q])
                    pltpu.async_copy(dst_hbm.at[pl.ds(noff, PSLAB)],
                                     dstv[1 - q], slsems[1 - q])

                for b in range(PNBUF):
                    pltpu.async_copy(gsp.at[srcv[q].at[b]], bufs[b],
                                     gsems[b])

                def group(gg, _):
                    for b in range(PNBUF):
                        j = gg * PNBUF + b
                        pltpu.make_async_copy(gsp.at[srcv[q].at[j]],
                                              bufs[b], gsems[b]).wait()
                        pltpu.async_copy(bufs[b], acc.at[dstv[q].at[j]],
                                         ssems[b], add=True)

                        @pl.when(j + PNBUF < PSLAB)
                        def _():
                            pltpu.make_async_copy(
                                bufs[b], acc.at[dstv[q].at[j]],
                                ssems[b]).wait()
                            pltpu.async_copy(gsp.at[srcv[q].at[j + PNBUF]],
                                             bufs[b], gsems[b])
                    return 0

                lax.fori_loop(0, PSLAB // PNBUF, group, 0)
                for b in range(PNBUF):
                    j = PSLAB - PNBUF + b
                    pltpu.make_async_copy(bufs[b], acc.at[dstv[q].at[j]],
                                          ssems[b]).wait()
            return 0

        lax.fori_loop(0, PNPH // 2, phase_pair, 0)
        plsc.subcore_barrier()
        pltpu.sync_copy(acc.at[pl.ds(s * PAPT, PAPT)],
                        out_hbm.at[h].at[c].at[pl.ds(s * PAPT, PAPT)])


_RB = 1000


def _tc1_body(deg_ref, x_ref, w1_ref, g1_ref, dinv_ref):
    deg = deg_ref[0, :, 0:1] + deg_ref[1, :, 0:1] + 1.0
    dinv = lax.rsqrt(deg)
    dinv_ref[...] = dinv
    g1_ref[...] = jnp.dot(x_ref[...], w1_ref[...],
                          preferred_element_type=jnp.float32) * dinv


def _tc1(degp, x, w1):
    return pl.pallas_call(
        _tc1_body,
        grid=(N // _RB,),
        in_specs=[
            pl.BlockSpec((NC, _RB, D), lambda i: (0, i, 0)),
            pl.BlockSpec((_RB, D), lambda i: (i, 0)),
            pl.BlockSpec((D, D), lambda i: (0, 0)),
        ],
        out_specs=[
            pl.BlockSpec((_RB, D), lambda i: (i, 0)),
            pl.BlockSpec((_RB, 1), lambda i: (i, 0)),
        ],
        out_shape=[
            jax.ShapeDtypeStruct((N, D), jnp.float32),
            jax.ShapeDtypeStruct((N, 1), jnp.float32),
        ],
    )(degp, x, w1)


def _tc2_body(p1_ref, g1_ref, dinv_ref, b1_ref, wcat_ref, g2_ref):
    dinv = dinv_ref[...]
    z = jnp.maximum(
        dinv * (p1_ref[0] + p1_ref[1] + g1_ref[...]) + b1_ref[...], 0.0)
    g2_ref[...] = jnp.dot(z, wcat_ref[...],
                          preferred_element_type=jnp.float32) * dinv


def _tc2(p1, g1, dinv, b1, wcat):
    return pl.pallas_call(
        _tc2_body,
        grid=(N // _RB,),
        in_specs=[
            pl.BlockSpec((NC, _RB, D), lambda i: (0, i, 0)),
            pl.BlockSpec((_RB, D), lambda i: (i, 0)),
            pl.BlockSpec((_RB, 1), lambda i: (i, 0)),
            pl.BlockSpec((1, D), lambda i: (0, 0)),
            pl.BlockSpec((D, D), lambda i: (0, 0)),
        ],
        out_specs=pl.BlockSpec((_RB, D), lambda i: (i, 0)),
        out_shape=jax.ShapeDtypeStruct((N, D), jnp.float32),
    )(p1, g1, dinv, b1, wcat)


def _tc3_body(p2_ref, g2_ref, dinv_ref, bcat_ref, out_ref):
    out_ref[...] = (dinv_ref[...] *
                    (p2_ref[0] + p2_ref[1] + g2_ref[...]) +
                    bcat_ref[...])


def _tc3(p2, g2, dinv, bcat):
    return pl.pallas_call(
        _tc3_body,
        grid=(N // _RB,),
        in_specs=[
            pl.BlockSpec((NC, _RB, D), lambda i: (0, i, 0)),
            pl.BlockSpec((_RB, D), lambda i: (i, 0)),
            pl.BlockSpec((_RB, 1), lambda i: (i, 0)),
            pl.BlockSpec((1, D), lambda i: (0, 0)),
        ],
        out_specs=pl.BlockSpec((_RB, D), lambda i: (i, 0)),
        out_shape=jax.ShapeDtypeStruct((N, D), jnp.float32),
    )(p2, g2, dinv, bcat)


def kernel(x, edge_index, W1, b1, W_mu, b_mu, W_ls, b_ls):
    E = edge_index.shape[1]
    pad = E_PAD - E
    src_p = jnp.concatenate([edge_index[0], jnp.zeros((pad,), jnp.int32)])
    dst_p = jnp.concatenate([edge_index[1], jnp.full((pad,), N, jnp.int32)])
    dstdeg2d = dst_p.reshape(NW * CPW, CHUNK)
    srcp2d = src_p.reshape(NW * PCPW, PCHUNK)
    dst0_2d = jnp.where(dst_p < HN, dst_p, TRASH).reshape(NW * PCPW, PCHUNK)
    dst1_2d = jnp.where((dst_p >= HN) & (dst_p < N), dst_p - HN,
                        TRASH).reshape(NW * PCPW, PCHUNK)

    onesD = jnp.ones((CHUNK, D), jnp.float32)
    zerosD = jnp.zeros((RPT, D), jnp.float32)
    wcat = jnp.concatenate([W_mu, W_ls], axis=1)
    bcat = jnp.concatenate([b_mu, b_ls]).reshape(1, D)

    degp = _sc_degree(dstdeg2d, onesD, zerosD)
    g1, dinv = _tc1(degp, x, W1)
    p1 = _sc_propagate(g1, srcp2d, dst0_2d, dst1_2d, zerosD)
    p1f = jnp.concatenate([p1[0, :, :HN], p1[1, :, :HN]], axis=1)
    g2 = _tc2(p1f, g1, dinv, b1.reshape(1, D), wcat)
    p2 = _sc_propagate(g2, srcp2d, dst0_2d, dst1_2d, zerosD)
    p2f = jnp.concatenate([p2[0, :, :HN], p2[1, :, :HN]], axis=1)
    out2 = _tc3(p2f, g2, dinv, bcat)
    return (out2[:, :OUT], out2[:, OUT:])


# speedup vs baseline: 14.6167x; 1.0350x over previous
import functools

import jax
import jax.numpy as jnp
from jax import lax
from jax.experimental import pallas as pl
from jax.experimental.pallas import tpu as pltpu
from jax.experimental.pallas import tpu_sc as plsc

N = 10000
D = 128
OUT = 64
NC = 2
NS = 16
NW = NC * NS
CHUNK = 64
CPW = 160
NBUF = 4
RPT = 632
ACC_ROWS = NS * RPT
E_PAD = NW * CPW * CHUNK

PCHUNK = 16
PCPW = E_PAD // (NW * PCHUNK)
PSLAB = 8
PNPH = PCPW // PSLAB
PNBUF = 2
HN = 5000
PAR = 5120
PAPT = PAR // NS
TRASH = 5100
GT = 10

_MESH = plsc.VectorSubcoreMesh(
    core_axis_name="c", subcore_axis_name="s", num_cores=NC, num_subcores=NS)


@functools.partial(
    pl.kernel,
    out_type=jax.ShapeDtypeStruct((NC, ACC_ROWS, D), jnp.float32),
    mesh=_MESH,
    scratch_types=[
        pltpu.VMEM((CPW, CHUNK), jnp.int32),
        pltpu.VMEM((CHUNK, D), jnp.float32),
        pltpu.VMEM_SHARED((ACC_ROWS, D), jnp.float32),
        [pltpu.SemaphoreType.DMA] * NBUF,
    ],
)
def _sc_degree(dst_hbm, ones_hbm, zeros_hbm, out_hbm, dstv, onesv, degsp,
               sems):
    c = lax.axis_index("c")
    s = lax.axis_index("s")
    wid = s * NC + c
    pltpu.sync_copy(dst_hbm.at[pl.ds(wid * CPW, CPW)], dstv)
    pltpu.sync_copy(ones_hbm, onesv)
    pltpu.sync_copy(zeros_hbm, degsp.at[pl.ds(s * RPT, RPT)])
    plsc.subcore_barrier()

    def body(g, _):
        for b in range(NBUF):
            pltpu.async_copy(onesv, degsp.at[dstv.at[g * NBUF + b]],
                             sems[b], add=True)
        for b in range(NBUF):
            pltpu.make_async_copy(
                onesv, degsp.at[dstv.at[g * NBUF + b]], sems[b]).wait()
        return 0

    lax.fori_loop(0, CPW // NBUF, body, 0)
    plsc.subcore_barrier()
    pltpu.sync_copy(degsp.at[pl.ds(s * RPT, RPT)],
                    out_hbm.at[c].at[pl.ds(s * RPT, RPT)])


@functools.partial(
    pl.kernel,
    out_type=jax.ShapeDtypeStruct((2, NC, PAR, D), jnp.float32),
    mesh=_MESH,
    scratch_types=[
        [pltpu.VMEM((PSLAB, PCHUNK), jnp.int32)] * 2,
        [pltpu.VMEM((PSLAB, PCHUNK), jnp.int32)] * 2,
        [pltpu.VMEM((PCHUNK, D), jnp.float32)] * PNBUF,
        pltpu.VMEM_SHARED((N, D), jnp.float32),
        pltpu.VMEM_SHARED((PAR, D), jnp.float32),
        [pltpu.SemaphoreType.DMA] * PNBUF,
        [pltpu.SemaphoreType.DMA] * PNBUF,
        [pltpu.SemaphoreType.DMA] * 2,
    ],
)
def _sc_propagate(g_hbm, src_hbm, dst0_hbm, dst1_hbm, zeros_hbm, out_hbm,
                  srcv, dstv, bufs, gsp, acc, gsems, ssems, slsems):
    c = lax.axis_index("c")
    s = lax.axis_index("s")
    wid = s * NC + c
    base = wid * PCPW

    @pl.when(s < GT)
    def _():
        pltpu.sync_copy(g_hbm.at[pl.ds(s * (N // GT), N // GT)],
                        gsp.at[pl.ds(s * (N // GT), N // GT)])

    for h in range(2):
        dst_hbm = dst0_hbm if h == 0 else dst1_hbm
        pltpu.sync_copy(zeros_hbm.at[pl.ds(0, PAPT)],
                        acc.at[pl.ds(s * PAPT, PAPT)])
        plsc.subcore_barrier()

        pltpu.async_copy(src_hbm.at[pl.ds(base, PSLAB)], srcv[0], slsems[0])
        pltpu.async_copy(dst_hbm.at[pl.ds(base, PSLAB)], dstv[0], slsems[0])

        def phase_pair(pp, _):
            for q in range(2):
                ph = pp * 2 + q
                off = base + ph * PSLAB
                pltpu.make_async_copy(src_hbm.at[pl.ds(off, PSLAB)],
                                      srcv[q], slsems[q]).wait()
                pltpu.make_async_copy(dst_hbm.at[pl.ds(off, PSLAB)],
                                      dstv[q], slsems[q]).wait()

                @pl.when(ph + 1 < PNPH)
                def _():
                    noff = base + (ph + 1) * PSLAB
                    pltpu.async_copy(src_hbm.at[pl.ds(noff, PSLAB)],
                                     srcv[1 - q], slsems[1 - q])
                    pltpu.async_copy(dst_hbm.at[pl.ds(noff, PSLAB)],
                                     dstv[1 - q], slsems[1 - q])

                for b in range(PNBUF):
                    pltpu.async_copy(gsp.at[srcv[q].at[b]], bufs[b],
                                     gsems[b])

                for j in range(PSLAB):
                    b = j % PNBUF
                    pltpu.make_async_copy(gsp.at[srcv[q].at[j]],
                                          bufs[b], gsems[b]).wait()
                    pltpu.async_copy(bufs[b], acc.at[dstv[q].at[j]],
                                     ssems[b], add=True)
                    if j + PNBUF < PSLAB:
                        pltpu.make_async_copy(bufs[b], acc.at[dstv[q].at[j]],
                                              ssems[b]).wait()
                        pltpu.async_copy(gsp.at[srcv[q].at[j + PNBUF]],
                                         bufs[b], gsems[b])
                for b in range(PNBUF):
                    j = PSLAB - PNBUF + b
                    pltpu.make_async_copy(bufs[b], acc.at[dstv[q].at[j]],
                                          ssems[b]).wait()
            return 0

        lax.fori_loop(0, PNPH // 2, phase_pair, 0)
        plsc.subcore_barrier()
        pltpu.sync_copy(acc.at[pl.ds(s * PAPT, PAPT)],
                        out_hbm.at[h].at[c].at[pl.ds(s * PAPT, PAPT)])


_RB = 1000


def _tc1_body(deg_ref, x_ref, w1_ref, g1_ref, dinv_ref):
    deg = deg_ref[0, :, 0:1] + deg_ref[1, :, 0:1] + 1.0
    dinv = lax.rsqrt(deg)
    dinv_ref[...] = dinv
    g1_ref[...] = jnp.dot(x_ref[...], w1_ref[...],
                          preferred_element_type=jnp.float32) * dinv


def _tc1(degp, x, w1):
    return pl.pallas_call(
        _tc1_body,
        grid=(N // _RB,),
        in_specs=[
            pl.BlockSpec((NC, _RB, D), lambda i: (0, i, 0)),
            pl.BlockSpec((_RB, D), lambda i: (i, 0)),
            pl.BlockSpec((D, D), lambda i: (0, 0)),
        ],
        out_specs=[
            pl.BlockSpec((_RB, D), lambda i: (i, 0)),
            pl.BlockSpec((_RB, 1), lambda i: (i, 0)),
        ],
        out_shape=[
            jax.ShapeDtypeStruct((N, D), jnp.float32),
            jax.ShapeDtypeStruct((N, 1), jnp.float32),
        ],
    )(degp, x, w1)


def _tc2_body(p1_ref, g1_ref, dinv_ref, b1_ref, wcat_ref, g2_ref):
    dinv = dinv_ref[...]
    z = jnp.maximum(
        dinv * (p1_ref[0, 0] + p1_ref[0, 1] + g1_ref[...]) + b1_ref[...], 0.0)
    g2_ref[...] = jnp.dot(z, wcat_ref[...],
                          preferred_element_type=jnp.float32) * dinv


_PHALF = pl.BlockSpec((1, NC, _RB, D), lambda i: (i // 5, 0, i % 5, 0))


def _tc2(p1, g1, dinv, b1, wcat):
    return pl.pallas_call(
        _tc2_body,
        grid=(N // _RB,),
        in_specs=[
            _PHALF,
            pl.BlockSpec((_RB, D), lambda i: (i, 0)),
            pl.BlockSpec((_RB, 1), lambda i: (i, 0)),
            pl.BlockSpec((1, D), lambda i: (0, 0)),
            pl.BlockSpec((D, D), lambda i: (0, 0)),
        ],
        out_specs=pl.BlockSpec((_RB, D), lambda i: (i, 0)),
        out_shape=jax.ShapeDtypeStruct((N, D), jnp.float32),
    )(p1, g1, dinv, b1, wcat)


def _tc3_body(p2_ref, g2_ref, dinv_ref, bcat_ref, out_ref):
    out_ref[...] = (dinv_ref[...] *
                    (p2_ref[0, 0] + p2_ref[0, 1] + g2_ref[...]) +
                    bcat_ref[...])


def _tc3(p2, g2, dinv, bcat):
    return pl.pallas_call(
        _tc3_body,
        grid=(N // _RB,),
        in_specs=[
            _PHALF,
            pl.BlockSpec((_RB, D), lambda i: (i, 0)),
            pl.BlockSpec((_RB, 1), lambda i: (i, 0)),
            pl.BlockSpec((1, D), lambda i: (0, 0)),
        ],
        out_specs=pl.BlockSpec((_RB, D), lambda i: (i, 0)),
        out_shape=jax.ShapeDtypeStruct((N, D), jnp.float32),
    )(p2, g2, dinv, bcat)


def kernel(x, edge_index, W1, b1, W_mu, b_mu, W_ls, b_ls):
    E = edge_index.shape[1]
    pad = E_PAD - E
    src_p = jnp.concatenate([edge_index[0], jnp.zeros((pad,), jnp.int32)])
    dst_p = jnp.concatenate([edge_index[1], jnp.full((pad,), N, jnp.int32)])
    dstdeg2d = dst_p.reshape(NW * CPW, CHUNK)
    srcp2d = src_p.reshape(NW * PCPW, PCHUNK)
    dst0_2d = jnp.where(dst_p < HN, dst_p, TRASH).reshape(NW * PCPW, PCHUNK)
    dst1_2d = jnp.where((dst_p >= HN) & (dst_p < N), dst_p - HN,
                        TRASH).reshape(NW * PCPW, PCHUNK)

    onesD = jnp.ones((CHUNK, D), jnp.float32)
    zerosD = jnp.zeros((RPT, D), jnp.float32)
    wcat = jnp.concatenate([W_mu, W_ls], axis=1)
    bcat = jnp.concatenate([b_mu, b_ls]).reshape(1, D)

    degp = _sc_degree(dstdeg2d, onesD, zerosD)
    g1, dinv = _tc1(degp, x, W1)
    p1 = _sc_propagate(g1, srcp2d, dst0_2d, dst1_2d, zerosD)
    g2 = _tc2(p1, g1, dinv, b1.reshape(1, D), wcat)
    p2 = _sc_propagate(g2, srcp2d, dst0_2d, dst1_2d, zerosD)
    out2 = _tc3(p2, g2, dinv, bcat)
    return (out2[:, :OUT], out2[:, OUT:])
